# speedup vs baseline: 1.3238x; 1.3238x over previous
import functools

import jax
import jax.numpy as jnp
from jax.experimental import pallas as pl
from jax.experimental.pallas import tpu as pltpu

K = 100
C = 1000
B = 1024
BLK = 128
NBLK = B // BLK


def _body(idx_ref, L_ref, P_ref, out_ref, W_ref, acc_ref):
    i = pl.program_id(0)

    @pl.when(i == 0)
    def _init():
        P = P_ref[...]
        m = jnp.max(P, axis=1, keepdims=True)
        e = jnp.exp(P - m)
        s = jnp.sum(e, axis=1, keepdims=True)
        W_ref[...] = e / s
        acc_ref[0] = 0.0

    idx = idx_ref[0, 0, :]
    onehot = (idx[:, None]
              == jax.lax.broadcasted_iota(jnp.int32, (BLK, K), 1)
              ).astype(jnp.float32)
    g = jnp.dot(onehot, W_ref[...], preferred_element_type=jnp.float32)
    acc_ref[0] += jnp.sum(L_ref[...] * g)

    @pl.when(i == pl.num_programs(0) - 1)
    def _fin():
        out_ref[0, 0] = acc_ref[0]


@functools.partial(jax.jit, static_argnames=("interpret",))
def _run(losses, inputs_idx, params, interpret=False):
    L = losses.reshape(B, C)
    idx3 = inputs_idx.astype(jnp.int32).reshape(NBLK, 1, BLK)
    out = pl.pallas_call(
        _body,
        grid=(NBLK,),
        in_specs=[
            pl.BlockSpec((1, 1, BLK), lambda i: (i, 0, 0)),
            pl.BlockSpec((BLK, C), lambda i: (i, 0)),
            pl.BlockSpec((K, C), lambda i: (0, 0)),
        ],
        out_specs=pl.BlockSpec(memory_space=pltpu.SMEM),
        out_shape=jax.ShapeDtypeStruct((1, 1), jnp.float32),
        scratch_shapes=[
            pltpu.VMEM((K, C), jnp.float32),
            pltpu.SMEM((1,), jnp.float32),
        ],
        interpret=interpret,
    )(idx3, L, params)
    return out[0, 0]


def kernel(losses, inputs_idx, params):
    return _run(losses, inputs_idx, params)
